# baseline (speedup 1.0000x reference)
import functools

import jax
import jax.numpy as jnp
from jax.experimental import pallas as pl

_N = 120
_TILE = 2000


_PREC = jax.lax.Precision.HIGHEST


def _dot(a, b):
    return jax.lax.dot_general(a, b, (((1,), (0,)), ((), ())),
                               precision=_PREC,
                               preferred_element_type=jnp.float32)


def _dott(a, b):
    return jax.lax.dot_general(a, b, (((0,), (0,)), ((), ())),
                               precision=_PREC,
                               preferred_element_type=jnp.float32)


def _relu(x):
    return jnp.maximum(x, 0.0)


def _flatten_params(params):
    out = []
    for blk in ('p1', 'p2', 'v1', 'v2'):
        mods = ('edge', 'node_mlp1', 'node_mlp2', 'global')
        if blk in ('p2', 'v2'):
            mods = ('edge', 'node_mlp1', 'node_mlp2')
        for m in mods:
            for (W, b) in params[blk][m]:
                out.append(W)
                out.append(b.reshape(1, -1))
    return out


def _body(nt, x0r, u0r, srcr, dstr, ear, *rest):
    prefs = list(rest[:-2])
    polr, valr = rest[-2:]

    vals = [r[...] for r in prefs]
    cursor = [0]

    def take(n):
        v = vals[cursor[0]:cursor[0] + n]
        cursor[0] += n
        return v

    blocks = {}
    for blk in ('p1', 'p2', 'v1', 'v2'):
        mods = ('edge', 'node_mlp1', 'node_mlp2', 'global')
        if blk in ('p2', 'v2'):
            mods = ('edge', 'node_mlp1', 'node_mlp2')
        d = {}
        for m in mods:
            d[m] = take(4)
        blocks[blk] = d

    x0 = x0r[...]
    u0 = u0r[...]

    def layer1_tables(blk):
        eW1, eb1, eW2, eb2 = blocks[blk]['edge']
        nW1, nb1, nW2, nb2 = blocks[blk]['node_mlp1']
        t = {}
        t['A'] = _dot(x0, eW1[0:5, :])
        t['B'] = _dot(x0, eW1[5:10, :])
        t['wc'] = eW1[10:11, :]
        t['U'] = _dot(u0, eW1[11:17, :]) + eb1
        t['C'] = _dot(x0, nW1[0:5, :])
        t['Nf'] = _dot(eW2, nW1[5:7, :])
        t['cf'] = nb1 + _dot(eb2, nW1[5:7, :])
        t['eW2'] = eW2
        t['eb2'] = eb2
        t['nW2'] = nW2
        t['nb2'] = nb2
        return t

    tp1 = layer1_tables('p1')
    tv1 = layer1_tables('v1')

    iota = jax.lax.broadcasted_iota(jnp.int32, (128, _TILE), 0)

    def onehots(t):
        srow = srcr[t]
        drow = dstr[t]
        ohs = (iota == srow).astype(jnp.float32)
        ohd = (iota == drow).astype(jnp.float32)
        return ohs, ohd

    def h1_of(t1, ohs, ohd, erow):
        return _relu(_dott(ohs, t1['A']) + _dott(ohd, t1['B'])
                     + _dott(erow, t1['wc']) + t1['U'])

    def pass1_body(t, carry):
        Sp, Sv, cnt = carry
        ohs, ohd = onehots(t)
        erow = ear[t]
        h1p = h1_of(tp1, ohs, ohd, erow)
        g1p = _relu(_dot(h1p, tp1['Nf']) + _dott(ohd, tp1['C']) + tp1['cf'])
        Sp = Sp + _dot(ohd, g1p)
        h1v = h1_of(tv1, ohs, ohd, erow)
        g1v = _relu(_dot(h1v, tv1['Nf']) + _dott(ohd, tv1['C']) + tv1['cf'])
        Sv = Sv + _dot(ohd, g1v)
        cnt = cnt + jnp.sum(ohd, axis=1, keepdims=True)
        return Sp, Sv, cnt

    zero128 = jnp.zeros((128, 128), jnp.float32)
    Sp, Sv, cnt = jax.lax.fori_loop(
        0, nt, pass1_body, (zero128, zero128, jnp.zeros((128, 1), jnp.float32)))

    cnt_safe = jnp.maximum(cnt, 1.0)
    mask = (jax.lax.broadcasted_iota(jnp.int32, (128, 1), 0) < _N).astype(jnp.float32)

    def node_stage(blk1, blk2, t1, S):
        n2W1, n2b1, n2W2, n2b2 = blocks[blk1]['node_mlp2']
        gW1, gb1, gW2, gb2 = blocks[blk1]['global']
        agg = _dot(S, t1['nW2']) + cnt * t1['nb2']
        aggm = agg / cnt_safe
        z = _relu(_dot(x0, n2W1[0:5, :]) + _dot(aggm, n2W1[5:133, :])
                  + cnt * n2W1[133:134, :] + n2b1)
        x1 = _dot(z, n2W2) + n2b2
        xm = jnp.sum(x1 * mask, axis=0, keepdims=True) * (1.0 / _N)
        u1 = _dot(_relu(_dot(u0, gW1[0:6, :]) + _dot(xm, gW1[6:16, :]) + gb1),
                  gW2) + gb2
        eW1, eb1, eW2, eb2 = blocks[blk2]['edge']
        mW1, mb1, mW2, mb2 = blocks[blk2]['node_mlp1']
        t2 = {}
        t2['A'] = _dot(x1, eW1[0:10, :])
        t2['B'] = _dot(x1, eW1[10:20, :])
        t2['M'] = _dot(t1['eW2'], eW1[20:22, :])
        t2['U'] = _dot(u1, eW1[22:34, :]) + eb1 + _dot(t1['eb2'], eW1[20:22, :])
        t2['C'] = _dot(x1, mW1[0:10, :])
        t2['Nf'] = _dot(eW2, mW1[10:11, :])
        t2['cf'] = mb1 + _dot(eb2, mW1[10:11, :])
        t2['mW2'] = mW2
        t2['mb2'] = mb2
        return x1, t2

    x1p, tp2 = node_stage('p1', 'p2', tp1, Sp)
    x1v, tv2 = node_stage('v1', 'v2', tv1, Sv)

    def pass2_body(t, carry):
        S2p, S2v = carry
        ohs, ohd = onehots(t)
        erow = ear[t]
        h1p = h1_of(tp1, ohs, ohd, erow)
        h2p = _relu(_dot(h1p, tp2['M']) + _dott(ohs, tp2['A'])
                    + _dott(ohd, tp2['B']) + tp2['U'])
        g2p = _relu(_dot(h2p, tp2['Nf']) + _dott(ohd, tp2['C']) + tp2['cf'])
        S2p = S2p + _dot(ohd, g2p)
        h1v = h1_of(tv1, ohs, ohd, erow)
        h2v = _relu(_dot(h1v, tv2['M']) + _dott(ohs, tv2['A'])
                    + _dott(ohd, tv2['B']) + tv2['U'])
        g2v = _relu(_dot(h2v, tv2['Nf']) + _dott(ohd, tv2['C']) + tv2['cf'])
        S2v = S2v + _dot(ohd, g2v)
        return S2p, S2v

    S2p, S2v = jax.lax.fori_loop(0, nt, pass2_body, (zero128, zero128))

    def final_stage(blk2, x1, t2, S2):
        q2W1, q2b1, q2W2, q2b2 = blocks[blk2]['node_mlp2']
        agg = _dot(S2, t2['mW2']) + cnt * t2['mb2']
        aggm = agg / cnt_safe
        z = _relu(_dot(x1, q2W1[0:10, :]) + _dot(aggm, q2W1[10:138, :])
                  + cnt * q2W1[138:139, :] + q2b1)
        return _dot(z, q2W2) + q2b2

    polr[...] = final_stage('p2', x1p, tp2, S2p)
    valr[...] = final_stage('v2', x1v, tv2, S2v)


def kernel(features, params):
    f = features[0]
    nodes = _N
    deg = f[0:nodes]
    cap = f[nodes:2 * nodes]
    inc = f[2 * nodes:3 * nodes]
    outg = f[3 * nodes:4 * nodes]
    tot = f[4 * nodes:5 * nodes]
    x0 = jnp.stack([cap, deg, inc, outg, tot], axis=1)
    x0 = jnp.pad(x0, ((0, 128 - nodes), (0, 0)))
    base = 5 * nodes + 6
    u0 = f[5 * nodes:base].reshape(1, 6)
    ne = (features.shape[1] - base) // 3
    nt = -(-ne // _TILE)
    pad = nt * _TILE - ne
    ea = f[base:base + ne]
    src = f[base + ne:base + 2 * ne].astype(jnp.int32)
    dst = f[base + 2 * ne:base + 3 * ne].astype(jnp.int32)
    if pad:
        ea = jnp.pad(ea, (0, pad))
        src = jnp.pad(src, (0, pad), constant_values=127)
        dst = jnp.pad(dst, (0, pad), constant_values=127)
    ea = ea.reshape(nt, 1, _TILE)
    src = src.reshape(nt, 1, _TILE)
    dst = dst.reshape(nt, 1, _TILE)

    plist = _flatten_params(params)
    pol, val = pl.pallas_call(
        functools.partial(_body, nt),
        out_shape=[jax.ShapeDtypeStruct((128, 1), jnp.float32),
                   jax.ShapeDtypeStruct((128, 1), jnp.float32)],
    )(x0, u0, src, dst, ea, *plist)
    policy = pol[:nodes, 0].reshape(1, nodes)
    value = val[:nodes, 0].reshape(1, nodes)
    return policy, value


# speedup vs baseline: 2.4503x; 2.4503x over previous
import functools

import jax
import jax.numpy as jnp
from jax.experimental import pallas as pl

_N = 120
_TILE = 2000
_F32 = jnp.float32
_BF16 = jnp.bfloat16


def _bf(x):
    return x.astype(_BF16)


def _dotbf(a, b):
    return jax.lax.dot_general(_bf(a), _bf(b), (((1,), (0,)), ((), ())),
                               preferred_element_type=_F32)


def _dot32(a, b):
    return jax.lax.dot_general(a, b, (((1,), (0,)), ((), ())),
                               precision=jax.lax.Precision.HIGHEST,
                               preferred_element_type=_F32)


def _dott(a, b):
    return jax.lax.dot_general(a, b, (((0,), (0,)), ((), ())),
                               preferred_element_type=_F32)


def _split(t):
    hi = _bf(t)
    lo = _bf(t - hi.astype(_F32))
    return hi, lo


def _gather(oh, pair):
    hi, lo = pair
    return _dott(oh, hi) + _dott(oh, lo)


def _rnd(x):
    return _bf(x).astype(_F32)


def _relu(x):
    return jnp.maximum(x, 0.0)


def _flatten_params(params):
    out = []
    for blk in ('p1', 'p2', 'v1', 'v2'):
        mods = ('edge', 'node_mlp1', 'node_mlp2', 'global')
        if blk in ('p2', 'v2'):
            mods = ('edge', 'node_mlp1', 'node_mlp2')
        for m in mods:
            for (W, b) in params[blk][m]:
                out.append(W)
                out.append(b.reshape(1, -1))
    return out


def _body(nt, x0r, u0r, srcr, dstr, ear, *rest):
    prefs = list(rest[:-2])
    polr, valr = rest[-2:]

    vals = [r[...] for r in prefs]
    cursor = [0]

    def take(n):
        v = vals[cursor[0]:cursor[0] + n]
        cursor[0] += n
        return v

    blocks = {}
    for blk in ('p1', 'p2', 'v1', 'v2'):
        mods = ('edge', 'node_mlp1', 'node_mlp2', 'global')
        if blk in ('p2', 'v2'):
            mods = ('edge', 'node_mlp1', 'node_mlp2')
        d = {}
        for m in mods:
            d[m] = take(4)
        blocks[blk] = d

    x0 = x0r[...]
    u0 = u0r[...]

    def layer1_tables(blk):
        eW1, eb1, eW2, eb2 = blocks[blk]['edge']
        nW1, nb1, nW2, nb2 = blocks[blk]['node_mlp1']
        t = {}
        t['A'] = _split(_dotbf(x0, eW1[0:5, :]))
        t['B'] = _split(_dotbf(x0, eW1[5:10, :]))
        t['wc'] = _bf(eW1[10:11, :])
        t['U'] = _dotbf(u0, eW1[11:17, :]) + eb1
        t['C'] = _split(_dotbf(x0, nW1[0:5, :]))
        t['eW2'] = _bf(eW2)
        t['eb2'] = eb2
        t['nWe'] = _bf(nW1[5:7, :])
        t['nb1'] = nb1
        t['nW2r'] = _rnd(nW2)
        t['nb2'] = nb2
        return t

    tp1 = layer1_tables('p1')
    tv1 = layer1_tables('v1')

    iota = jax.lax.broadcasted_iota(jnp.int32, (128, _TILE), 0)

    def onehots(t):
        srow = srcr[t]
        drow = dstr[t]
        ohs = (iota == srow).astype(_BF16)
        ohd = (iota == drow).astype(_BF16)
        return ohs, ohd

    def edge1(t1, ohs, ohd, erow):
        h1 = _relu(_gather(ohs, t1['A']) + _gather(ohd, t1['B'])
                   + _dott(_bf(erow), t1['wc']) + t1['U'])
        e1 = jax.lax.dot_general(_bf(h1), t1['eW2'], (((1,), (0,)), ((), ())),
                                 preferred_element_type=_F32) + t1['eb2']
        return h1, e1

    def gcomp(t1, ohd, e1):
        m = _relu(_gather(ohd, t1['C'])
                  + jax.lax.dot_general(_bf(e1), t1['nWe'],
                                        (((1,), (0,)), ((), ())),
                                        preferred_element_type=_F32)
                  + t1['nb1'])
        return _bf(m)

    def pass1_body(t, carry):
        Sp, Sv, cnt = carry
        ohs, ohd = onehots(t)
        erow = ear[t]
        _, e1p = edge1(tp1, ohs, ohd, erow)
        Sp = Sp + jax.lax.dot_general(ohd, gcomp(tp1, ohd, e1p),
                                      (((1,), (0,)), ((), ())),
                                      preferred_element_type=_F32)
        _, e1v = edge1(tv1, ohs, ohd, erow)
        Sv = Sv + jax.lax.dot_general(ohd, gcomp(tv1, ohd, e1v),
                                      (((1,), (0,)), ((), ())),
                                      preferred_element_type=_F32)
        cnt = cnt + jnp.sum(ohd.astype(_F32), axis=1, keepdims=True)
        return Sp, Sv, cnt

    zero128 = jnp.zeros((128, 128), _F32)
    Sp, Sv, cnt = jax.lax.fori_loop(
        0, nt, pass1_body, (zero128, zero128, jnp.zeros((128, 1), _F32)))

    cnt_safe = jnp.maximum(cnt, 1.0)
    cnt_r = _rnd(cnt)
    mask = (jax.lax.broadcasted_iota(jnp.int32, (128, 1), 0) < _N).astype(_F32)

    def node_stage(blk1, blk2, t1, S):
        n2W1, n2b1, n2W2, n2b2 = blocks[blk1]['node_mlp2']
        gW1, gb1, gW2, gb2 = blocks[blk1]['global']
        agg = _dot32(S, t1['nW2r']) + cnt * t1['nb2']
        aggm = agg / cnt_safe
        z = _relu(_dotbf(x0, n2W1[0:5, :]) + _dotbf(aggm, n2W1[5:133, :])
                  + cnt_r * _rnd(n2W1[133:134, :]) + n2b1)
        x1 = _dotbf(z, n2W2) + n2b2
        xm = jnp.sum(x1 * mask, axis=0, keepdims=True) * (1.0 / _N)
        u1 = _dotbf(_relu(_dotbf(u0, gW1[0:6, :]) + _dotbf(xm, gW1[6:16, :])
                          + gb1), gW2) + gb2
        eW1, eb1, eW2, eb2 = blocks[blk2]['edge']
        mW1, mb1, mW2, mb2 = blocks[blk2]['node_mlp1']
        t2 = {}
        t2['A'] = _split(_dotbf(x1, eW1[0:10, :]))
        t2['B'] = _split(_dotbf(x1, eW1[10:20, :]))
        t2['We'] = _bf(eW1[20:22, :])
        t2['U'] = _dotbf(u1, eW1[22:34, :]) + eb1
        t2['C'] = _split(_dotbf(x1, mW1[0:10, :]))
        t2['eW2'] = _bf(eW2)
        t2['eb2'] = eb2
        t2['nWe'] = _bf(mW1[10:11, :])
        t2['nb1'] = mb1
        t2['nW2r'] = _rnd(mW2)
        t2['nb2'] = mb2
        return x1, t2

    x1p, tp2 = node_stage('p1', 'p2', tp1, Sp)
    x1v, tv2 = node_stage('v1', 'v2', tv1, Sv)

    def edge2(t1, t2, ohs, ohd, erow):
        _, e1 = edge1(t1, ohs, ohd, erow)
        h2 = _relu(_gather(ohs, t2['A']) + _gather(ohd, t2['B'])
                   + jax.lax.dot_general(_bf(e1), t2['We'],
                                         (((1,), (0,)), ((), ())),
                                         preferred_element_type=_F32)
                   + t2['U'])
        e2 = jax.lax.dot_general(_bf(h2), t2['eW2'], (((1,), (0,)), ((), ())),
                                 preferred_element_type=_F32) + t2['eb2']
        return gcomp(t2, ohd, e2)

    def pass2_body(t, carry):
        S2p, S2v = carry
        ohs, ohd = onehots(t)
        erow = ear[t]
        S2p = S2p + jax.lax.dot_general(ohd, edge2(tp1, tp2, ohs, ohd, erow),
                                        (((1,), (0,)), ((), ())),
                                        preferred_element_type=_F32)
        S2v = S2v + jax.lax.dot_general(ohd, edge2(tv1, tv2, ohs, ohd, erow),
                                        (((1,), (0,)), ((), ())),
                                        preferred_element_type=_F32)
        return S2p, S2v

    S2p, S2v = jax.lax.fori_loop(0, nt, pass2_body, (zero128, zero128))

    def final_stage(blk2, x1, t2, S2):
        q2W1, q2b1, q2W2, q2b2 = blocks[blk2]['node_mlp2']
        agg = _dot32(S2, t2['nW2r']) + cnt * t2['nb2']
        aggm = agg / cnt_safe
        z = _relu(_dotbf(x1, q2W1[0:10, :]) + _dotbf(aggm, q2W1[10:138, :])
                  + cnt_r * _rnd(q2W1[138:139, :]) + q2b1)
        return _dotbf(z, q2W2) + q2b2

    polr[...] = final_stage('p2', x1p, tp2, S2p)
    valr[...] = final_stage('v2', x1v, tv2, S2v)


def kernel(features, params):
    f = features[0]
    nodes = _N
    deg = f[0:nodes]
    cap = f[nodes:2 * nodes]
    inc = f[2 * nodes:3 * nodes]
    outg = f[3 * nodes:4 * nodes]
    tot = f[4 * nodes:5 * nodes]
    x0 = jnp.stack([cap, deg, inc, outg, tot], axis=1)
    x0 = jnp.pad(x0, ((0, 128 - nodes), (0, 0)))
    base = 5 * nodes + 6
    u0 = f[5 * nodes:base].reshape(1, 6)
    ne = (features.shape[1] - base) // 3
    nt = -(-ne // _TILE)
    pad = nt * _TILE - ne
    ea = f[base:base + ne]
    src = f[base + ne:base + 2 * ne].astype(jnp.int32)
    dst = f[base + 2 * ne:base + 3 * ne].astype(jnp.int32)
    if pad:
        ea = jnp.pad(ea, (0, pad))
        src = jnp.pad(src, (0, pad), constant_values=127)
        dst = jnp.pad(dst, (0, pad), constant_values=127)
    ea = ea.reshape(nt, 1, _TILE)
    src = src.reshape(nt, 1, _TILE)
    dst = dst.reshape(nt, 1, _TILE)

    plist = _flatten_params(params)
    pol, val = pl.pallas_call(
        functools.partial(_body, nt),
        out_shape=[jax.ShapeDtypeStruct((128, 1), _F32),
                   jax.ShapeDtypeStruct((128, 1), _F32)],
    )(x0, u0, src, dst, ea, *plist)
    policy = pol[:nodes, 0].reshape(1, nodes)
    value = val[:nodes, 0].reshape(1, nodes)
    return policy, value


# speedup vs baseline: 4.3797x; 1.7874x over previous
import functools

import jax
import jax.numpy as jnp
from jax.experimental import pallas as pl

_N = 120
_TILE = 2000
_F32 = jnp.float32
_BF16 = jnp.bfloat16


def _bf(x):
    return x.astype(_BF16)


def _dotbf(a, b):
    return jax.lax.dot_general(_bf(a), _bf(b), (((1,), (0,)), ((), ())),
                               preferred_element_type=_F32)


def _dot32(a, b):
    return jax.lax.dot_general(a, b, (((1,), (0,)), ((), ())),
                               precision=jax.lax.Precision.HIGHEST,
                               preferred_element_type=_F32)


def _dott(a, b):
    return jax.lax.dot_general(a, b, (((0,), (0,)), ((), ())),
                               preferred_element_type=_F32)


def _split(t):
    hi = _bf(t)
    lo = _bf(t - hi.astype(_F32))
    return hi, lo


def _gather(oh, pair):
    hi, lo = pair
    return _dott(oh, hi) + _dott(oh, lo)


def _rnd(x):
    return _bf(x).astype(_F32)


def _relu(x):
    return jnp.maximum(x, 0.0)


def _flatten_params(params):
    out = []
    for blk in ('p1', 'p2', 'v1', 'v2'):
        mods = ('edge', 'node_mlp1', 'node_mlp2', 'global')
        if blk in ('p2', 'v2'):
            mods = ('edge', 'node_mlp1', 'node_mlp2')
        for m in mods:
            for (W, b) in params[blk][m]:
                out.append(W)
                out.append(b.reshape(1, -1))
    return out


def _body(nt, x0r, u0r, srcr, dstr, ear, *rest):
    prefs = list(rest[:-2])
    polr, valr = rest[-2:]

    vals = [r[...] for r in prefs]
    cursor = [0]

    def take(n):
        v = vals[cursor[0]:cursor[0] + n]
        cursor[0] += n
        return v

    blocks = {}
    for blk in ('p1', 'p2', 'v1', 'v2'):
        mods = ('edge', 'node_mlp1', 'node_mlp2', 'global')
        if blk in ('p2', 'v2'):
            mods = ('edge', 'node_mlp1', 'node_mlp2')
        d = {}
        for m in mods:
            d[m] = take(4)
        blocks[blk] = d

    x0 = x0r[...]
    u0 = u0r[...]

    def layer1_tables(blk):
        eW1, eb1, eW2, eb2 = blocks[blk]['edge']
        nW1, nb1, nW2, nb2 = blocks[blk]['node_mlp1']
        t = {}
        t['A'] = _split(_dotbf(x0, eW1[0:5, :]))
        t['B'] = _split(_dotbf(x0, eW1[5:10, :]))
        t['wc'] = _bf(eW1[10:11, :])
        t['U'] = _dotbf(u0, eW1[11:17, :]) + eb1
        t['C'] = _split(_dotbf(x0, nW1[0:5, :]))
        t['eW2'] = _bf(eW2)
        t['eb2'] = eb2
        t['nWe'] = _bf(nW1[5:7, :])
        t['nb1'] = nb1
        t['nW2r'] = _rnd(nW2)
        t['nb2'] = nb2
        return t

    tp1 = layer1_tables('p1')
    tv1 = layer1_tables('v1')

    iota = jax.lax.broadcasted_iota(jnp.int32, (128, _TILE), 0)

    def onehots(t):
        srow = srcr[t]
        drow = dstr[t]
        ohs = (iota == srow).astype(_BF16)
        ohd = (iota == drow).astype(_BF16)
        return ohs, ohd

    def sl(g, i):
        return g[:, 128 * i:128 * (i + 1)]

    def _mxu(a, b):
        return jax.lax.dot_general(a, b, (((1,), (0,)), ((), ())),
                                   preferred_element_type=_F32)

    def eterm(e, rows):
        acc = _rnd(e[:, 0:1]) * rows[0]
        for i in range(1, len(rows)):
            acc = acc + _rnd(e[:, i:i + 1]) * rows[i]
        return acc

    def _rows32(w):
        return [w[i:i + 1, :].astype(_F32) for i in range(w.shape[0])]

    for t1 in (tp1, tv1):
        t1['nWe_r'] = _rows32(t1['nWe'])

    OHS1 = jnp.concatenate([tp1['A'][0], tp1['A'][1],
                            tv1['A'][0], tv1['A'][1]], axis=1)
    OHD1 = jnp.concatenate([tp1['B'][0], tp1['B'][1], tp1['C'][0], tp1['C'][1],
                            tv1['B'][0], tv1['B'][1], tv1['C'][0], tv1['C'][1]],
                           axis=1)
    EA1 = jnp.concatenate([tp1['wc'], tv1['wc']], axis=1)

    def edges1(gs, ge, gBp, gBv):
        h1p = _relu(sl(gs, 0) + sl(gs, 1) + gBp + sl(ge, 0) + tp1['U'])
        h1v = _relu(sl(gs, 2) + sl(gs, 3) + gBv + sl(ge, 1) + tv1['U'])
        e1p = _mxu(_bf(h1p), tp1['eW2']) + tp1['eb2']
        e1v = _mxu(_bf(h1v), tv1['eW2']) + tv1['eb2']
        return e1p, e1v

    def pass1_body(t, carry):
        S, cnt = carry
        ohs, ohd = onehots(t)
        erow = ear[t]
        gs = _dott(ohs, OHS1)
        gd = _dott(ohd, OHD1)
        ge = _dott(_bf(erow), EA1)
        e1p, e1v = edges1(gs, ge, sl(gd, 0) + sl(gd, 1), sl(gd, 4) + sl(gd, 5))
        gp = _bf(_relu(sl(gd, 2) + sl(gd, 3) + eterm(e1p, tp1['nWe_r'])
                       + tp1['nb1']))
        gv = _bf(_relu(sl(gd, 6) + sl(gd, 7) + eterm(e1v, tv1['nWe_r'])
                       + tv1['nb1']))
        S = S + _mxu(ohd, jnp.concatenate([gp, gv], axis=1))
        cnt = cnt + jnp.sum(ohd.astype(_F32), axis=1, keepdims=True)
        return S, cnt

    S1, cnt = jax.lax.fori_loop(
        0, nt, pass1_body,
        (jnp.zeros((128, 256), _F32), jnp.zeros((128, 1), _F32)))
    Sp, Sv = S1[:, 0:128], S1[:, 128:256]

    cnt_safe = jnp.maximum(cnt, 1.0)
    cnt_r = _rnd(cnt)
    mask = (jax.lax.broadcasted_iota(jnp.int32, (128, 1), 0) < _N).astype(_F32)

    def node_stage(blk1, blk2, t1, S):
        n2W1, n2b1, n2W2, n2b2 = blocks[blk1]['node_mlp2']
        gW1, gb1, gW2, gb2 = blocks[blk1]['global']
        agg = _dot32(S, t1['nW2r']) + cnt * t1['nb2']
        aggm = agg / cnt_safe
        z = _relu(_dotbf(x0, n2W1[0:5, :]) + _dotbf(aggm, n2W1[5:133, :])
                  + cnt_r * _rnd(n2W1[133:134, :]) + n2b1)
        x1 = _dotbf(z, n2W2) + n2b2
        xm = jnp.sum(x1 * mask, axis=0, keepdims=True) * (1.0 / _N)
        u1 = _dotbf(_relu(_dotbf(u0, gW1[0:6, :]) + _dotbf(xm, gW1[6:16, :])
                          + gb1), gW2) + gb2
        eW1, eb1, eW2, eb2 = blocks[blk2]['edge']
        mW1, mb1, mW2, mb2 = blocks[blk2]['node_mlp1']
        t2 = {}
        t2['A'] = _split(_dotbf(x1, eW1[0:10, :]))
        t2['B'] = _split(_dotbf(x1, eW1[10:20, :]))
        t2['We'] = _bf(eW1[20:22, :])
        t2['U'] = _dotbf(u1, eW1[22:34, :]) + eb1
        t2['C'] = _split(_dotbf(x1, mW1[0:10, :]))
        t2['eW2'] = _bf(eW2)
        t2['eb2'] = eb2
        t2['nWe'] = _bf(mW1[10:11, :])
        t2['nb1'] = mb1
        t2['nW2r'] = _rnd(mW2)
        t2['nb2'] = mb2
        return x1, t2

    x1p, tp2 = node_stage('p1', 'p2', tp1, Sp)
    x1v, tv2 = node_stage('v1', 'v2', tv1, Sv)

    for t2 in (tp2, tv2):
        t2['We_r'] = _rows32(t2['We'])
        t2['nWe_r'] = _rows32(t2['nWe'])

    OHS2 = jnp.concatenate([OHS1,
                            tp2['A'][0], tp2['A'][1],
                            tv2['A'][0], tv2['A'][1]], axis=1)
    OHD2 = jnp.concatenate([tp1['B'][0], tp1['B'][1],
                            tv1['B'][0], tv1['B'][1],
                            tp2['B'][0], tp2['B'][1], tp2['C'][0], tp2['C'][1],
                            tv2['B'][0], tv2['B'][1], tv2['C'][0], tv2['C'][1]],
                           axis=1)

    def pass2_body(t, carry):
        S2 = carry
        ohs, ohd = onehots(t)
        erow = ear[t]
        gs = _dott(ohs, OHS2)
        gd = _dott(ohd, OHD2)
        ge = _dott(_bf(erow), EA1)
        e1p, e1v = edges1(gs, ge, sl(gd, 0) + sl(gd, 1), sl(gd, 2) + sl(gd, 3))
        h2p = _relu(sl(gs, 4) + sl(gs, 5) + sl(gd, 4) + sl(gd, 5)
                    + eterm(e1p, tp2['We_r']) + tp2['U'])
        h2v = _relu(sl(gs, 6) + sl(gs, 7) + sl(gd, 8) + sl(gd, 9)
                    + eterm(e1v, tv2['We_r']) + tv2['U'])
        e2p = _mxu(_bf(h2p), tp2['eW2']) + tp2['eb2']
        e2v = _mxu(_bf(h2v), tv2['eW2']) + tv2['eb2']
        g2p = _bf(_relu(sl(gd, 6) + sl(gd, 7) + eterm(e2p, tp2['nWe_r'])
                        + tp2['nb1']))
        g2v = _bf(_relu(sl(gd, 10) + sl(gd, 11) + eterm(e2v, tv2['nWe_r'])
                        + tv2['nb1']))
        return S2 + _mxu(ohd, jnp.concatenate([g2p, g2v], axis=1))

    S2 = jax.lax.fori_loop(0, nt, pass2_body, jnp.zeros((128, 256), _F32))
    S2p, S2v = S2[:, 0:128], S2[:, 128:256]

    def final_stage(blk2, x1, t2, S2):
        q2W1, q2b1, q2W2, q2b2 = blocks[blk2]['node_mlp2']
        agg = _dot32(S2, t2['nW2r']) + cnt * t2['nb2']
        aggm = agg / cnt_safe
        z = _relu(_dotbf(x1, q2W1[0:10, :]) + _dotbf(aggm, q2W1[10:138, :])
                  + cnt_r * _rnd(q2W1[138:139, :]) + q2b1)
        return _dotbf(z, q2W2) + q2b2

    polr[...] = final_stage('p2', x1p, tp2, S2p)
    valr[...] = final_stage('v2', x1v, tv2, S2v)


def kernel(features, params):
    f = features[0]
    nodes = _N
    deg = f[0:nodes]
    cap = f[nodes:2 * nodes]
    inc = f[2 * nodes:3 * nodes]
    outg = f[3 * nodes:4 * nodes]
    tot = f[4 * nodes:5 * nodes]
    x0 = jnp.stack([cap, deg, inc, outg, tot], axis=1)
    x0 = jnp.pad(x0, ((0, 128 - nodes), (0, 0)))
    base = 5 * nodes + 6
    u0 = f[5 * nodes:base].reshape(1, 6)
    ne = (features.shape[1] - base) // 3
    nt = -(-ne // _TILE)
    pad = nt * _TILE - ne
    ea = f[base:base + ne]
    src = f[base + ne:base + 2 * ne].astype(jnp.int32)
    dst = f[base + 2 * ne:base + 3 * ne].astype(jnp.int32)
    if pad:
        ea = jnp.pad(ea, (0, pad))
        src = jnp.pad(src, (0, pad), constant_values=127)
        dst = jnp.pad(dst, (0, pad), constant_values=127)
    ea = ea.reshape(nt, 1, _TILE)
    src = src.reshape(nt, 1, _TILE)
    dst = dst.reshape(nt, 1, _TILE)

    plist = _flatten_params(params)
    pol, val = pl.pallas_call(
        functools.partial(_body, nt),
        out_shape=[jax.ShapeDtypeStruct((128, 1), _F32),
                   jax.ShapeDtypeStruct((128, 1), _F32)],
    )(x0, u0, src, dst, ea, *plist)
    policy = pol[:nodes, 0].reshape(1, nodes)
    value = val[:nodes, 0].reshape(1, nodes)
    return policy, value


# speedup vs baseline: 5.3420x; 1.2197x over previous
import functools

import jax
import jax.numpy as jnp
from jax.experimental import pallas as pl
from jax.experimental.pallas import tpu as pltpu

_N = 120
_TILE = 2000
_F32 = jnp.float32
_BF16 = jnp.bfloat16


def _bf(x):
    return x.astype(_BF16)


def _dotbf(a, b):
    return jax.lax.dot_general(_bf(a), _bf(b), (((1,), (0,)), ((), ())),
                               preferred_element_type=_F32)


def _dot32(a, b):
    return jax.lax.dot_general(a, b, (((1,), (0,)), ((), ())),
                               precision=jax.lax.Precision.HIGHEST,
                               preferred_element_type=_F32)


def _dott(a, b):
    return jax.lax.dot_general(a, b, (((0,), (0,)), ((), ())),
                               preferred_element_type=_F32)


def _split(t):
    hi = _bf(t)
    lo = _bf(t - hi.astype(_F32))
    return hi, lo


def _gather(oh, pair):
    hi, lo = pair
    return _dott(oh, hi) + _dott(oh, lo)


def _rnd(x):
    return _bf(x).astype(_F32)


def _relu(x):
    return jnp.maximum(x, 0.0)


def _flatten_params(params):
    out = []
    for blk in ('p1', 'p2', 'v1', 'v2'):
        mods = ('edge', 'node_mlp1', 'node_mlp2', 'global')
        if blk in ('p2', 'v2'):
            mods = ('edge', 'node_mlp1', 'node_mlp2')
        for m in mods:
            for (W, b) in params[blk][m]:
                out.append(W)
                out.append(b.reshape(1, -1))
    return out


def _body(nt, x0r, u0r, srcr, dstr, ear, *rest):
    prefs = list(rest[:-3])
    polr, valr, e1r = rest[-3:]

    vals = [r[...] for r in prefs]
    cursor = [0]

    def take(n):
        v = vals[cursor[0]:cursor[0] + n]
        cursor[0] += n
        return v

    blocks = {}
    for blk in ('p1', 'p2', 'v1', 'v2'):
        mods = ('edge', 'node_mlp1', 'node_mlp2', 'global')
        if blk in ('p2', 'v2'):
            mods = ('edge', 'node_mlp1', 'node_mlp2')
        d = {}
        for m in mods:
            d[m] = take(4)
        blocks[blk] = d

    x0 = x0r[...]
    u0 = u0r[...]

    def layer1_tables(blk):
        eW1, eb1, eW2, eb2 = blocks[blk]['edge']
        nW1, nb1, nW2, nb2 = blocks[blk]['node_mlp1']
        t = {}
        t['A'] = _split(_dotbf(x0, eW1[0:5, :]))
        t['B'] = _split(_dotbf(x0, eW1[5:10, :]))
        t['wc'] = _bf(eW1[10:11, :])
        t['U'] = _dotbf(u0, eW1[11:17, :]) + eb1
        t['C'] = _split(_dotbf(x0, nW1[0:5, :]))
        t['eW2'] = _bf(eW2)
        t['eb2'] = eb2
        t['nWe'] = _bf(nW1[5:7, :])
        t['nb1'] = nb1
        t['nW2r'] = _rnd(nW2)
        t['nb2'] = nb2
        return t

    tp1 = layer1_tables('p1')
    tv1 = layer1_tables('v1')

    iota = jax.lax.broadcasted_iota(jnp.int32, (128, _TILE), 0)

    def onehots(t):
        srow = srcr[t]
        drow = dstr[t]
        ohs = (iota == srow).astype(_BF16)
        ohd = (iota == drow).astype(_BF16)
        return ohs, ohd

    def sl(g, i):
        return g[:, 128 * i:128 * (i + 1)]

    def _mxu(a, b):
        return jax.lax.dot_general(a, b, (((1,), (0,)), ((), ())),
                                   preferred_element_type=_F32)

    def eterm(e, rows):
        acc = _rnd(e[:, 0:1]) * rows[0]
        for i in range(1, len(rows)):
            acc = acc + _rnd(e[:, i:i + 1]) * rows[i]
        return acc

    def _rows32(w):
        return [w[i:i + 1, :].astype(_F32) for i in range(w.shape[0])]

    for t1 in (tp1, tv1):
        t1['nWe_r'] = _rows32(t1['nWe'])

    OHS1 = jnp.concatenate([tp1['A'][0], tp1['A'][1],
                            tv1['A'][0], tv1['A'][1]], axis=1)
    OHD1 = jnp.concatenate([tp1['B'][0], tp1['B'][1], tp1['C'][0], tp1['C'][1],
                            tv1['B'][0], tv1['B'][1], tv1['C'][0], tv1['C'][1]],
                           axis=1)
    EA1 = jnp.concatenate([tp1['wc'], tv1['wc']], axis=1)

    def edges1(gs, ge, gBp, gBv):
        h1p = _relu(sl(gs, 0) + sl(gs, 1) + gBp + sl(ge, 0) + tp1['U'])
        h1v = _relu(sl(gs, 2) + sl(gs, 3) + gBv + sl(ge, 1) + tv1['U'])
        e1p = _mxu(_bf(h1p), tp1['eW2']) + tp1['eb2']
        e1v = _mxu(_bf(h1v), tv1['eW2']) + tv1['eb2']
        return e1p, e1v

    def pass1_body(t, carry):
        S, cnt = carry
        ohs, ohd = onehots(t)
        erow = ear[t]
        gs = _dott(ohs, OHS1)
        gd = _dott(ohd, OHD1)
        ge = _dott(_bf(erow), EA1)
        e1p, e1v = edges1(gs, ge, sl(gd, 0) + sl(gd, 1), sl(gd, 4) + sl(gd, 5))
        e1r[pl.ds(t * _TILE, _TILE), :] = _bf(jnp.concatenate([e1p, e1v],
                                                              axis=1))
        gp = _bf(_relu(sl(gd, 2) + sl(gd, 3) + eterm(e1p, tp1['nWe_r'])
                       + tp1['nb1']))
        gv = _bf(_relu(sl(gd, 6) + sl(gd, 7) + eterm(e1v, tv1['nWe_r'])
                       + tv1['nb1']))
        S = S + _mxu(ohd, jnp.concatenate([gp, gv], axis=1))
        cnt = cnt + jnp.sum(ohd.astype(_F32), axis=1, keepdims=True)
        return S, cnt

    S1, cnt = jax.lax.fori_loop(
        0, nt, pass1_body,
        (jnp.zeros((128, 256), _F32), jnp.zeros((128, 1), _F32)))
    Sp, Sv = S1[:, 0:128], S1[:, 128:256]

    cnt_safe = jnp.maximum(cnt, 1.0)
    cnt_r = _rnd(cnt)
    mask = (jax.lax.broadcasted_iota(jnp.int32, (128, 1), 0) < _N).astype(_F32)

    def node_stage(blk1, blk2, t1, S):
        n2W1, n2b1, n2W2, n2b2 = blocks[blk1]['node_mlp2']
        gW1, gb1, gW2, gb2 = blocks[blk1]['global']
        agg = _dot32(S, t1['nW2r']) + cnt * t1['nb2']
        aggm = agg / cnt_safe
        z = _relu(_dotbf(x0, n2W1[0:5, :]) + _dotbf(aggm, n2W1[5:133, :])
                  + cnt_r * _rnd(n2W1[133:134, :]) + n2b1)
        x1 = _dotbf(z, n2W2) + n2b2
        xm = jnp.sum(x1 * mask, axis=0, keepdims=True) * (1.0 / _N)
        u1 = _dotbf(_relu(_dotbf(u0, gW1[0:6, :]) + _dotbf(xm, gW1[6:16, :])
                          + gb1), gW2) + gb2
        eW1, eb1, eW2, eb2 = blocks[blk2]['edge']
        mW1, mb1, mW2, mb2 = blocks[blk2]['node_mlp1']
        t2 = {}
        t2['A'] = _split(_dotbf(x1, eW1[0:10, :]))
        t2['B'] = _split(_dotbf(x1, eW1[10:20, :]))
        t2['We'] = _bf(eW1[20:22, :])
        t2['U'] = _dotbf(u1, eW1[22:34, :]) + eb1
        t2['C'] = _split(_dotbf(x1, mW1[0:10, :]))
        t2['eW2'] = _bf(eW2)
        t2['eb2'] = eb2
        t2['nWe'] = _bf(mW1[10:11, :])
        t2['nb1'] = mb1
        t2['nW2r'] = _rnd(mW2)
        t2['nb2'] = mb2
        return x1, t2

    x1p, tp2 = node_stage('p1', 'p2', tp1, Sp)
    x1v, tv2 = node_stage('v1', 'v2', tv1, Sv)

    for t2 in (tp2, tv2):
        t2['We_r'] = _rows32(t2['We'])
        t2['nWe_r'] = _rows32(t2['nWe'])

    OHS2 = jnp.concatenate([tp2['A'][0], tp2['A'][1],
                            tv2['A'][0], tv2['A'][1]], axis=1)
    OHD2 = jnp.concatenate([tp2['B'][0], tp2['B'][1], tp2['C'][0], tp2['C'][1],
                            tv2['B'][0], tv2['B'][1], tv2['C'][0], tv2['C'][1]],
                           axis=1)

    def pass2_body(t, carry):
        S2 = carry
        ohs, ohd = onehots(t)
        gs = _dott(ohs, OHS2)
        gd = _dott(ohd, OHD2)
        ec = e1r[pl.ds(t * _TILE, _TILE), :]
        h2p = _relu(sl(gs, 0) + sl(gs, 1) + sl(gd, 0) + sl(gd, 1)
                    + eterm(ec[:, 0:2], tp2['We_r']) + tp2['U'])
        h2v = _relu(sl(gs, 2) + sl(gs, 3) + sl(gd, 4) + sl(gd, 5)
                    + eterm(ec[:, 2:4], tv2['We_r']) + tv2['U'])
        e2p = _mxu(_bf(h2p), tp2['eW2']) + tp2['eb2']
        e2v = _mxu(_bf(h2v), tv2['eW2']) + tv2['eb2']
        g2p = _bf(_relu(sl(gd, 2) + sl(gd, 3) + eterm(e2p, tp2['nWe_r'])
                        + tp2['nb1']))
        g2v = _bf(_relu(sl(gd, 6) + sl(gd, 7) + eterm(e2v, tv2['nWe_r'])
                        + tv2['nb1']))
        return S2 + _mxu(ohd, jnp.concatenate([g2p, g2v], axis=1))

    S2 = jax.lax.fori_loop(0, nt, pass2_body, jnp.zeros((128, 256), _F32))
    S2p, S2v = S2[:, 0:128], S2[:, 128:256]

    def final_stage(blk2, x1, t2, S2):
        q2W1, q2b1, q2W2, q2b2 = blocks[blk2]['node_mlp2']
        agg = _dot32(S2, t2['nW2r']) + cnt * t2['nb2']
        aggm = agg / cnt_safe
        z = _relu(_dotbf(x1, q2W1[0:10, :]) + _dotbf(aggm, q2W1[10:138, :])
                  + cnt_r * _rnd(q2W1[138:139, :]) + q2b1)
        return _dotbf(z, q2W2) + q2b2

    polr[...] = final_stage('p2', x1p, tp2, S2p)
    valr[...] = final_stage('v2', x1v, tv2, S2v)


def kernel(features, params):
    f = features[0]
    nodes = _N
    deg = f[0:nodes]
    cap = f[nodes:2 * nodes]
    inc = f[2 * nodes:3 * nodes]
    outg = f[3 * nodes:4 * nodes]
    tot = f[4 * nodes:5 * nodes]
    x0 = jnp.stack([cap, deg, inc, outg, tot], axis=1)
    x0 = jnp.pad(x0, ((0, 128 - nodes), (0, 0)))
    base = 5 * nodes + 6
    u0 = f[5 * nodes:base].reshape(1, 6)
    ne = (features.shape[1] - base) // 3
    nt = -(-ne // _TILE)
    pad = nt * _TILE - ne
    ea = f[base:base + ne]
    src = f[base + ne:base + 2 * ne].astype(jnp.int32)
    dst = f[base + 2 * ne:base + 3 * ne].astype(jnp.int32)
    if pad:
        ea = jnp.pad(ea, (0, pad))
        src = jnp.pad(src, (0, pad), constant_values=127)
        dst = jnp.pad(dst, (0, pad), constant_values=127)
    ea = ea.reshape(nt, 1, _TILE)
    src = src.reshape(nt, 1, _TILE)
    dst = dst.reshape(nt, 1, _TILE)

    plist = _flatten_params(params)
    pol, val = pl.pallas_call(
        functools.partial(_body, nt),
        out_shape=[jax.ShapeDtypeStruct((128, 1), _F32),
                   jax.ShapeDtypeStruct((128, 1), _F32)],
        scratch_shapes=[pltpu.VMEM((nt * _TILE, 4), _BF16)],
    )(x0, u0, src, dst, ea, *plist)
    policy = pol[:nodes, 0].reshape(1, nodes)
    value = val[:nodes, 0].reshape(1, nodes)
    return policy, value


# speedup vs baseline: 6.2665x; 1.1731x over previous
import functools

import jax
import jax.numpy as jnp
from jax.experimental import pallas as pl
from jax.experimental.pallas import tpu as pltpu

_N = 120
_TILE = 2000
_F32 = jnp.float32
_BF16 = jnp.bfloat16


def _bf(x):
    return x.astype(_BF16)


def _dotbf(a, b):
    return jax.lax.dot_general(_bf(a), _bf(b), (((1,), (0,)), ((), ())),
                               preferred_element_type=_F32)


def _dot32(a, b):
    return jax.lax.dot_general(a, b, (((1,), (0,)), ((), ())),
                               precision=jax.lax.Precision.HIGHEST,
                               preferred_element_type=_F32)


def _dott(a, b):
    return jax.lax.dot_general(a, b, (((0,), (0,)), ((), ())),
                               preferred_element_type=_F32)


def _split(t):
    hi = _bf(t)
    lo = _bf(t - hi.astype(_F32))
    return jnp.concatenate([hi, lo], axis=0)


def _rnd(x):
    return _bf(x).astype(_F32)


def _relu(x):
    return jnp.maximum(x, 0.0)


def _flatten_params(params):
    out = []
    for blk in ('p1', 'p2', 'v1', 'v2'):
        mods = ('edge', 'node_mlp1', 'node_mlp2', 'global')
        if blk in ('p2', 'v2'):
            mods = ('edge', 'node_mlp1', 'node_mlp2')
        for m in mods:
            for (W, b) in params[blk][m]:
                out.append(W)
                out.append(b.reshape(1, -1))
    return out


def _body(nt, x0r, u0r, srcr, dstr, ear, *rest):
    prefs = list(rest[:-3])
    polr, valr, e1r = rest[-3:]

    vals = [r[...] for r in prefs]
    cursor = [0]

    def take(n):
        v = vals[cursor[0]:cursor[0] + n]
        cursor[0] += n
        return v

    blocks = {}
    for blk in ('p1', 'p2', 'v1', 'v2'):
        mods = ('edge', 'node_mlp1', 'node_mlp2', 'global')
        if blk in ('p2', 'v2'):
            mods = ('edge', 'node_mlp1', 'node_mlp2')
        d = {}
        for m in mods:
            d[m] = take(4)
        blocks[blk] = d

    x0 = x0r[...]
    u0 = u0r[...]

    def layer1_tables(blk):
        eW1, eb1, eW2, eb2 = blocks[blk]['edge']
        nW1, nb1, nW2, nb2 = blocks[blk]['node_mlp1']
        t = {}
        t['A'] = _split(_dotbf(x0, eW1[0:5, :]))
        t['B'] = _split(_dotbf(x0, eW1[5:10, :]))
        t['wc'] = _bf(eW1[10:11, :])
        t['U'] = _split(_dotbf(u0, eW1[11:17, :]) + eb1)
        t['C'] = _split(_dotbf(x0, nW1[0:5, :]))
        t['eW2'] = _bf(eW2)
        t['eb2'] = eb2
        t['nWe'] = _bf(nW1[5:7, :])
        t['nb1'] = nb1
        t['nW2r'] = _rnd(nW2)
        t['nb2'] = nb2
        return t

    tp1 = layer1_tables('p1')
    tv1 = layer1_tables('v1')

    iota2 = jnp.bitwise_and(
        jax.lax.broadcasted_iota(jnp.int32, (256, _TILE), 0), 127)

    def onehots(t):
        srow = srcr[t]
        drow = dstr[t]
        ohs2 = (iota2 == srow).astype(_BF16)
        ohd2 = (iota2 == drow).astype(_BF16)
        return ohs2, ohd2

    def sl(g, i):
        return g[:, 128 * i:128 * (i + 1)]

    def _mxu(a, b):
        return jax.lax.dot_general(a, b, (((1,), (0,)), ((), ())),
                                   preferred_element_type=_F32)

    def eterm(e, rows):
        acc = _rnd(e[:, 0:1]) * rows[0]
        for i in range(1, len(rows)):
            acc = acc + _rnd(e[:, i:i + 1]) * rows[i]
        return acc

    def _rows32(w):
        return [w[i:i + 1, :].astype(_F32) for i in range(w.shape[0])]

    for t1 in (tp1, tv1):
        t1['nWe_r'] = _rows32(t1['nWe'])

    OHS1 = jnp.concatenate([tp1['A'], tv1['A']], axis=1)
    OHD1 = jnp.concatenate([tp1['B'], tp1['C'],
                            tv1['B'], tv1['C']], axis=1)
    EAU1 = jnp.concatenate([
        jnp.concatenate([tp1['wc'], tv1['wc']], axis=1),
        jnp.concatenate([tp1['U'][0:1], tv1['U'][0:1]], axis=1),
        jnp.concatenate([tp1['U'][1:2], tv1['U'][1:2]], axis=1)], axis=0)
    ones_row = jnp.ones((2, _TILE), _BF16)
    ones_col = jnp.ones((_TILE, 1), _BF16)

    def edges1(gs, ge, gBp, gBv):
        h1p = _relu(sl(gs, 0) + gBp + sl(ge, 0))
        h1v = _relu(sl(gs, 1) + gBv + sl(ge, 1))
        e1p = _mxu(_bf(h1p), tp1['eW2']) + tp1['eb2']
        e1v = _mxu(_bf(h1v), tv1['eW2']) + tv1['eb2']
        return e1p, e1v

    def pass1_body(t, carry):
        S, cnt = carry
        ohs2, ohd2 = onehots(t)
        erow = ear[t]
        gs = _dott(ohs2, OHS1)
        gd = _dott(ohd2, OHD1)
        ge = _dott(jnp.concatenate([_bf(erow), ones_row], axis=0), EAU1)
        e1p, e1v = edges1(gs, ge, sl(gd, 0), sl(gd, 2))
        e1r[pl.ds(t * _TILE, _TILE), :] = _bf(jnp.concatenate([e1p, e1v],
                                                              axis=1))
        gp = _bf(_relu(sl(gd, 1) + eterm(e1p, tp1['nWe_r']) + tp1['nb1']))
        gv = _bf(_relu(sl(gd, 3) + eterm(e1v, tv1['nWe_r']) + tv1['nb1']))
        ohd = ohd2[0:128, :]
        S = S + _mxu(ohd, jnp.concatenate([gp, gv], axis=1))
        cnt = cnt + _mxu(ohd, ones_col)
        return S, cnt

    S1, cnt = jax.lax.fori_loop(
        0, nt, pass1_body,
        (jnp.zeros((128, 256), _F32), jnp.zeros((128, 1), _F32)))
    Sp, Sv = S1[:, 0:128], S1[:, 128:256]

    cnt_safe = jnp.maximum(cnt, 1.0)
    cnt_r = _rnd(cnt)
    mask = (jax.lax.broadcasted_iota(jnp.int32, (128, 1), 0) < _N).astype(_F32)

    def node_stage(blk1, blk2, t1, S):
        n2W1, n2b1, n2W2, n2b2 = blocks[blk1]['node_mlp2']
        gW1, gb1, gW2, gb2 = blocks[blk1]['global']
        agg = _dot32(S, t1['nW2r']) + cnt * t1['nb2']
        aggm = agg / cnt_safe
        z = _relu(_dotbf(x0, n2W1[0:5, :]) + _dotbf(aggm, n2W1[5:133, :])
                  + cnt_r * _rnd(n2W1[133:134, :]) + n2b1)
        x1 = _dotbf(z, n2W2) + n2b2
        xm = jnp.sum(x1 * mask, axis=0, keepdims=True) * (1.0 / _N)
        u1 = _dotbf(_relu(_dotbf(u0, gW1[0:6, :]) + _dotbf(xm, gW1[6:16, :])
                          + gb1), gW2) + gb2
        eW1, eb1, eW2, eb2 = blocks[blk2]['edge']
        mW1, mb1, mW2, mb2 = blocks[blk2]['node_mlp1']
        t2 = {}
        t2['A'] = _split(_dotbf(x1, eW1[0:10, :]))
        t2['B'] = _split(_dotbf(x1, eW1[10:20, :]))
        t2['We'] = _bf(eW1[20:22, :])
        t2['U'] = _dotbf(u1, eW1[22:34, :]) + eb1
        t2['C'] = _split(_dotbf(x1, mW1[0:10, :]))
        t2['eW2'] = _bf(eW2)
        t2['eb2'] = eb2
        t2['nWe'] = _bf(mW1[10:11, :])
        t2['nb1'] = mb1
        t2['nW2r'] = _rnd(mW2)
        t2['nb2'] = mb2
        return x1, t2

    x1p, tp2 = node_stage('p1', 'p2', tp1, Sp)
    x1v, tv2 = node_stage('v1', 'v2', tv1, Sv)

    for t2 in (tp2, tv2):
        t2['We_r'] = _rows32(t2['We'])
        t2['nWe_r'] = _rows32(t2['nWe'])

    OHS2 = jnp.concatenate([tp2['A'], tv2['A']], axis=1)
    OHD2 = jnp.concatenate([tp2['B'], tp2['C'],
                            tv2['B'], tv2['C']], axis=1)

    def pass2_body(t, carry):
        S2 = carry
        ohs2, ohd2 = onehots(t)
        gs = _dott(ohs2, OHS2)
        gd = _dott(ohd2, OHD2)
        ec = e1r[pl.ds(t * _TILE, _TILE), :]
        h2p = _relu(sl(gs, 0) + sl(gd, 0)
                    + eterm(ec[:, 0:2], tp2['We_r']) + tp2['U'])
        h2v = _relu(sl(gs, 1) + sl(gd, 2)
                    + eterm(ec[:, 2:4], tv2['We_r']) + tv2['U'])
        e2p = _mxu(_bf(h2p), tp2['eW2']) + tp2['eb2']
        e2v = _mxu(_bf(h2v), tv2['eW2']) + tv2['eb2']
        g2p = _bf(_relu(sl(gd, 1) + eterm(e2p, tp2['nWe_r']) + tp2['nb1']))
        g2v = _bf(_relu(sl(gd, 3) + eterm(e2v, tv2['nWe_r']) + tv2['nb1']))
        return S2 + _mxu(ohd2[0:128, :], jnp.concatenate([g2p, g2v], axis=1))

    S2 = jax.lax.fori_loop(0, nt, pass2_body, jnp.zeros((128, 256), _F32))
    S2p, S2v = S2[:, 0:128], S2[:, 128:256]

    def final_stage(blk2, x1, t2, S2):
        q2W1, q2b1, q2W2, q2b2 = blocks[blk2]['node_mlp2']
        agg = _dot32(S2, t2['nW2r']) + cnt * t2['nb2']
        aggm = agg / cnt_safe
        z = _relu(_dotbf(x1, q2W1[0:10, :]) + _dotbf(aggm, q2W1[10:138, :])
                  + cnt_r * _rnd(q2W1[138:139, :]) + q2b1)
        return _dotbf(z, q2W2) + q2b2

    polr[...] = final_stage('p2', x1p, tp2, S2p)
    valr[...] = final_stage('v2', x1v, tv2, S2v)


def kernel(features, params):
    f = features[0]
    nodes = _N
    deg = f[0:nodes]
    cap = f[nodes:2 * nodes]
    inc = f[2 * nodes:3 * nodes]
    outg = f[3 * nodes:4 * nodes]
    tot = f[4 * nodes:5 * nodes]
    x0 = jnp.stack([cap, deg, inc, outg, tot], axis=1)
    x0 = jnp.pad(x0, ((0, 128 - nodes), (0, 0)))
    base = 5 * nodes + 6
    u0 = f[5 * nodes:base].reshape(1, 6)
    ne = (features.shape[1] - base) // 3
    nt = -(-ne // _TILE)
    pad = nt * _TILE - ne
    ea = f[base:base + ne]
    src = f[base + ne:base + 2 * ne].astype(jnp.int32)
    dst = f[base + 2 * ne:base + 3 * ne].astype(jnp.int32)
    if pad:
        ea = jnp.pad(ea, (0, pad))
        src = jnp.pad(src, (0, pad), constant_values=127)
        dst = jnp.pad(dst, (0, pad), constant_values=127)
    ea = ea.reshape(nt, 1, _TILE)
    src = src.reshape(nt, 1, _TILE)
    dst = dst.reshape(nt, 1, _TILE)

    plist = _flatten_params(params)
    pol, val = pl.pallas_call(
        functools.partial(_body, nt),
        out_shape=[jax.ShapeDtypeStruct((128, 1), _F32),
                   jax.ShapeDtypeStruct((128, 1), _F32)],
        scratch_shapes=[pltpu.VMEM((nt * _TILE, 4), _BF16)],
    )(x0, u0, src, dst, ea, *plist)
    policy = pol[:nodes, 0].reshape(1, nodes)
    value = val[:nodes, 0].reshape(1, nodes)
    return policy, value
